# baseline (speedup 1.0000x reference)
import functools

import jax
import jax.numpy as jnp
from jax import lax
from jax.experimental import pallas as pl
from jax.experimental.pallas import tpu as pltpu
from jax.experimental.pallas import tpu_sc as plsc

_EPS = 0.1
_IGNORE = 0

_NC = 2
_NS = 16
_NW = _NC * _NS
_LANES = 16


def _sc_gather_body(flat_hbm, lab_hbm, out_hbm, lab_v, idx_v, g_v, sem, *, k, rows_per_w):
    wid = lax.axis_index("s") * _NC + lax.axis_index("c")
    base = wid * rows_per_w
    pltpu.sync_copy(lab_hbm.at[pl.ds(base, rows_per_w)], lab_v)
    lanes = lax.iota(jnp.int32, _LANES)
    for j in range(rows_per_w // _LANES):
        row = base + j * _LANES + lanes
        lab = lab_v[pl.ds(j * _LANES, _LANES)]
        idx_v[pl.ds(j * _LANES, _LANES)] = row * k + lab
    pltpu.async_copy(flat_hbm.at[idx_v], g_v, sem).wait()
    pltpu.sync_copy(g_v, out_hbm.at[pl.ds(base, rows_per_w)])


def _sc_gather(preds_flat, labels_i32, k):
    r = labels_i32.shape[0]
    rows_per_w = r // _NW
    mesh = plsc.VectorSubcoreMesh(core_axis_name="c", subcore_axis_name="s")
    return pl.kernel(
        functools.partial(_sc_gather_body, k=k, rows_per_w=rows_per_w),
        out_type=jax.ShapeDtypeStruct((r,), jnp.float32),
        mesh=mesh,
        scratch_types=[
            pltpu.VMEM((rows_per_w,), jnp.int32),
            pltpu.VMEM((rows_per_w,), jnp.int32),
            pltpu.VMEM((rows_per_w,), jnp.float32),
            pltpu.SemaphoreType.DMA,
        ],
    )(preds_flat, labels_i32)


def _ce_body(x_ref, lab_ref, g_ref, num_ref, den_ref):
    i = pl.program_id(0)
    x = x_ref[...]
    lab = lab_ref[0, 0, :]
    g = g_ref[0, 0, :]
    k = x.shape[1]

    m = jnp.max(x, axis=1, keepdims=True)
    s = jnp.sum(jnp.exp(x - m), axis=1, keepdims=True)
    t = jnp.sum(x, axis=1, keepdims=True)
    lse = m + jnp.log(s)

    mask = (lab[:, None] != _IGNORE).astype(x.dtype)
    per = lse - (1.0 - _EPS) * g[:, None] - (_EPS / k) * t
    pnum = jnp.sum(per * mask, axis=0, keepdims=True)
    pden = jnp.sum(mask, axis=0, keepdims=True)

    @pl.when(i == 0)
    def _init():
        num_ref[...] = jnp.zeros_like(num_ref)
        den_ref[...] = jnp.zeros_like(den_ref)

    num_ref[...] += pnum
    den_ref[...] += pden


@functools.partial(jax.jit, static_argnames=("rb",))
def _ce_loss(preds, labels, rb=32):
    r, k = preds.shape
    nb = r // rb
    labels_i32 = labels.astype(jnp.int32)
    g = _sc_gather(preds.reshape(-1), labels_i32, k)
    lab3 = labels_i32.reshape(nb, 1, rb)
    g3 = g.reshape(nb, 1, rb)
    num, den = pl.pallas_call(
        _ce_body,
        grid=(nb,),
        in_specs=[
            pl.BlockSpec((rb, k), lambda i: (i, 0)),
            pl.BlockSpec((1, 1, rb), lambda i: (i, 0, 0)),
            pl.BlockSpec((1, 1, rb), lambda i: (i, 0, 0)),
        ],
        out_specs=[
            pl.BlockSpec((1, 1), lambda i: (0, 0)),
            pl.BlockSpec((1, 1), lambda i: (0, 0)),
        ],
        out_shape=[
            jax.ShapeDtypeStruct((1, 1), preds.dtype),
            jax.ShapeDtypeStruct((1, 1), preds.dtype),
        ],
    )(preds, lab3, g3)
    return num[0, 0] / den[0, 0]


def kernel(preds, labels):
    return _ce_loss(preds, labels)


# speedup vs baseline: 1.3427x; 1.3427x over previous
import functools

import jax
import jax.numpy as jnp
from jax import lax
from jax.experimental import pallas as pl
from jax.experimental.pallas import tpu as pltpu
from jax.experimental.pallas import tpu_sc as plsc

_EPS = 0.1
_IGNORE = 0

_NC = 2
_NS = 16
_NW = _NC * _NS
_L = 16

_R_SC = 256
_CW = 1408
_NCHUNK = 71
_KTAIL = 32
_NEG = -3.0e38


def _sc_row_pass(buf, r, cw, c0, lab_v, iota, macc, sacc, tacc, g_ref):
    nv = cw // _L

    def amax_body(j, carry):
        mv, tv = carry
        v = buf[r, pl.ds(j * _L, _L)]
        return jnp.maximum(mv, v), tv + v

    mv_c, tv = lax.fori_loop(
        0, nv, amax_body,
        (jnp.full((_L,), _NEG, jnp.float32), jnp.zeros((_L,), jnp.float32)),
    )
    m_old = macc[r]
    m_new = jnp.maximum(m_old, mv_c)
    rescale = jnp.exp(m_old - m_new)
    macc[r] = m_new
    tacc[r] = tacc[r] + tv

    def exp_body(j, sv):
        v = buf[r, pl.ds(j * _L, _L)]
        return sv + jnp.exp(v - m_new)

    sv = lax.fori_loop(0, nv, exp_body, jnp.zeros((_L,), jnp.float32))
    sacc[r] = sacc[r] * rescale + sv


def _sc_chunk(buf, cw, c0, lab_vec, iota, macc, sacc, tacc, gacc):
    nv = cw // _L
    for r in range(8):
        _sc_row_pass(buf, r, cw, c0, lab_vec, iota, macc, sacc, tacc, gacc)
        lab_r = lab_vec[r]
        in_win = (lab_r >= c0) & (lab_r < c0 + cw)

        @pl.when(in_win)
        def _pick():
            off = jnp.full((_L,), lab_r - c0, jnp.int32)

            def gbody(j, gv):
                v = buf[r, pl.ds(j * _L, _L)]
                hit = (iota + j * _L) == off
                return gv + jnp.where(hit, v, 0.0)

            gacc[r] = gacc[r] + lax.fori_loop(
                0, nv, gbody, jnp.zeros((_L,), jnp.float32))


def _sc_dense_body(x_hbm, lab_hbm, m_hbm, s_hbm, t_hbm, g_hbm,
                   buf, tbuf, lab_v, macc, sacc, tacc, gacc, sem):
    wid = lax.axis_index("s") * _NC + lax.axis_index("c")
    r0 = wid * 8
    iota = lax.iota(jnp.int32, _L)

    pltpu.sync_copy(lab_hbm.at[pl.ds(r0, _L)], lab_v)
    lab_vec = lab_v[...]

    for r in range(8):
        macc[r] = jnp.full((_L,), _NEG, jnp.float32)
        sacc[r] = jnp.zeros((_L,), jnp.float32)
        tacc[r] = jnp.zeros((_L,), jnp.float32)
        gacc[r] = jnp.zeros((_L,), jnp.float32)

    def chunk_body(ci, _):
        c0 = ci * _CW
        pltpu.sync_copy(x_hbm.at[pl.ds(r0, 8), pl.ds(c0, _CW)], buf)
        _sc_chunk(buf, _CW, c0, lab_vec, iota, macc, sacc, tacc, gacc)
        return 0

    lax.fori_loop(0, _NCHUNK, chunk_body, 0)

    c0 = _NCHUNK * _CW
    pltpu.sync_copy(x_hbm.at[pl.ds(r0, 8), pl.ds(c0, _KTAIL)], tbuf)
    _sc_chunk(tbuf, _KTAIL, c0, lab_vec, iota, macc, sacc, tacc, gacc)

    pltpu.sync_copy(macc, m_hbm.at[pl.ds(r0, 8)])
    pltpu.sync_copy(sacc, s_hbm.at[pl.ds(r0, 8)])
    pltpu.sync_copy(tacc, t_hbm.at[pl.ds(r0, 8)])
    pltpu.sync_copy(gacc, g_hbm.at[pl.ds(r0, 8)])


def _sc_dense(preds, labels_i32):
    mesh = plsc.VectorSubcoreMesh(core_axis_name="c", subcore_axis_name="s")
    per_row = jax.ShapeDtypeStruct((_R_SC, _L), jnp.float32)
    return pl.kernel(
        _sc_dense_body,
        out_type=(per_row, per_row, per_row, per_row),
        mesh=mesh,
        scratch_types=[
            pltpu.VMEM((8, _CW), jnp.float32),
            pltpu.VMEM((8, _KTAIL), jnp.float32),
            pltpu.VMEM((_L,), jnp.int32),
            pltpu.VMEM((8, _L), jnp.float32),
            pltpu.VMEM((8, _L), jnp.float32),
            pltpu.VMEM((8, _L), jnp.float32),
            pltpu.VMEM((8, _L), jnp.float32),
            pltpu.SemaphoreType.DMA,
        ],
    )(preds, labels_i32)


def _tc_body(x_ref, lab_ref, num_ref, den_ref):
    i = pl.program_id(0)
    x = x_ref[...]
    lab = lab_ref[0, 0, :]
    k = x.shape[1]

    m = jnp.max(x, axis=1, keepdims=True)
    s = jnp.sum(jnp.exp(x - m), axis=1, keepdims=True)
    lse = m + jnp.log(s)

    cols = jax.lax.broadcasted_iota(jnp.int32, x.shape, 1)
    w = jnp.where(cols == lab[:, None], (1.0 - _EPS) + _EPS / k, _EPS / k)
    wx = jnp.sum(w * x, axis=1, keepdims=True)

    mask = (lab[:, None] != _IGNORE).astype(x.dtype)
    per = lse - wx
    pnum = jnp.sum(per * mask, axis=0, keepdims=True)
    pden = jnp.sum(mask, axis=0, keepdims=True)

    @pl.when(i == 0)
    def _init():
        num_ref[...] = jnp.zeros_like(num_ref)
        den_ref[...] = jnp.zeros_like(den_ref)

    num_ref[...] += pnum
    den_ref[...] += pden


def _combine_body(num_ref, den_ref, m_ref, s_ref, t_ref, g_ref, lab_ref,
                  out_ref, *, k):
    ml = m_ref[...]
    m = jnp.max(ml, axis=1, keepdims=True)
    s = jnp.sum(s_ref[...] * jnp.exp(ml - m), axis=1, keepdims=True)
    t = jnp.sum(t_ref[...], axis=1, keepdims=True)
    g = jnp.sum(g_ref[...], axis=1, keepdims=True)
    lab = lab_ref[0, 0, :][:, None]
    lse = m + jnp.log(s)
    per = lse - (1.0 - _EPS) * g - (_EPS / k) * t
    mask = (lab != _IGNORE).astype(jnp.float32)
    num = num_ref[0, 0] + jnp.sum(per * mask)
    den = den_ref[0, 0] + jnp.sum(mask)
    out_ref[...] = jnp.full((1, 1), num / den, jnp.float32)


@functools.partial(jax.jit, static_argnames=("rb",))
def _ce_loss(preds, labels, rb=32):
    r, k = preds.shape
    labels_i32 = labels.astype(jnp.int32)

    m_sc, s_sc, t_sc, g_sc = _sc_dense(preds, labels_i32)

    nb_sc = _R_SC // rb
    nb = (r - _R_SC) // rb
    lab3 = labels_i32.reshape(r // rb, 1, rb)
    num, den = pl.pallas_call(
        _tc_body,
        grid=(nb,),
        in_specs=[
            pl.BlockSpec((rb, k), lambda i: (i + nb_sc, 0)),
            pl.BlockSpec((1, 1, rb), lambda i: (i + nb_sc, 0, 0)),
        ],
        out_specs=[
            pl.BlockSpec((1, 1), lambda i: (0, 0)),
            pl.BlockSpec((1, 1), lambda i: (0, 0)),
        ],
        out_shape=[
            jax.ShapeDtypeStruct((1, 1), preds.dtype),
            jax.ShapeDtypeStruct((1, 1), preds.dtype),
        ],
    )(preds, lab3)

    lab_sc3 = labels_i32[:_R_SC].reshape(1, 1, _R_SC)
    loss = pl.pallas_call(
        functools.partial(_combine_body, k=k),
        out_shape=jax.ShapeDtypeStruct((1, 1), jnp.float32),
    )(num, den, m_sc, s_sc, t_sc, g_sc, lab_sc3)
    return loss[0, 0]


def kernel(preds, labels):
    return _ce_loss(preds, labels)


# speedup vs baseline: 5.7146x; 4.2560x over previous
import functools

import jax
import jax.numpy as jnp
from jax.experimental import pallas as pl
from jax.experimental.pallas import tpu as pltpu

_EPS = 0.1
_IGNORE = 0
_NEG = -3.0e38


def _tct_body(x_ref, lab_ref, num_ref, den_ref, m_s, s_s, t_s, g_s, *, ck, nb):
    i = pl.program_id(0)
    x = x_ref[...]
    lab = lab_ref[0]
    r = x.shape[1]

    @pl.when(i == 0)
    def _init():
        m_s[...] = jnp.full_like(m_s, _NEG)
        s_s[...] = jnp.zeros_like(s_s)
        t_s[...] = jnp.zeros_like(t_s)
        g_s[...] = jnp.zeros_like(g_s)

    m_old = m_s[...]
    mc = jnp.max(x, axis=0, keepdims=True)
    m_new = jnp.maximum(m_old, mc)
    corr = jnp.exp(m_old - m_new)
    s_new = s_s[...] * corr + jnp.sum(jnp.exp(x - m_new), axis=0, keepdims=True)
    m_s[...] = m_new
    s_s[...] = s_new
    t_s[...] += jnp.sum(x, axis=0, keepdims=True)

    rows = jax.lax.broadcasted_iota(jnp.int32, x.shape, 0) + i * ck
    hit = rows == lab
    g_new = g_s[...] + jnp.sum(jnp.where(hit, x, 0.0), axis=0, keepdims=True)
    g_s[...] = g_new

    @pl.when(i == nb - 1)
    def _fin():
        k = nb * ck
        lse = m_new + jnp.log(s_new)
        per = lse - (1.0 - _EPS) * g_new - (_EPS / k) * t_s[...]
        mask = (lab != _IGNORE).astype(jnp.float32)
        num_ref[...] = jnp.sum(per * mask).reshape(1, 1)
        den_ref[...] = jnp.sum(mask).reshape(1, 1)


@functools.partial(jax.jit, static_argnames=("ck",))
def _ce_loss(preds, labels, ck=2000):
    r, k = preds.shape
    pt = preds.T
    nb = k // ck
    lab2 = labels.astype(jnp.int32).reshape(1, r)
    num, den = pl.pallas_call(
        functools.partial(_tct_body, ck=ck, nb=nb),
        grid=(nb,),
        in_specs=[
            pl.BlockSpec((ck, r), lambda i: (i, 0)),
            pl.BlockSpec((1, r), lambda i: (0, 0)),
        ],
        out_specs=[
            pl.BlockSpec((1, 1), lambda i: (0, 0)),
            pl.BlockSpec((1, 1), lambda i: (0, 0)),
        ],
        out_shape=[
            jax.ShapeDtypeStruct((1, 1), jnp.float32),
            jax.ShapeDtypeStruct((1, 1), jnp.float32),
        ],
        scratch_shapes=[
            pltpu.VMEM((1, r), jnp.float32),
            pltpu.VMEM((1, r), jnp.float32),
            pltpu.VMEM((1, r), jnp.float32),
            pltpu.VMEM((1, r), jnp.float32),
        ],
    )(pt, lab2)
    return num[0, 0] / den[0, 0]


def kernel(preds, labels):
    return _ce_loss(preds, labels)


# speedup vs baseline: 5.7367x; 1.0039x over previous
import functools

import jax
import jax.numpy as jnp
from jax.experimental import pallas as pl
from jax.experimental.pallas import tpu as pltpu

_EPS = 0.1
_IGNORE = 0
_NEG = -3.0e38


def _tct_body(x_ref, lab_ref, num_ref, den_ref, m_s, s_s, t_s, g_s, *, ck, nb):
    i = pl.program_id(0)
    x = x_ref[...]
    lab = lab_ref[0]
    r = x.shape[1]

    @pl.when(i == 0)
    def _init():
        m_s[...] = jnp.full_like(m_s, _NEG)
        s_s[...] = jnp.zeros_like(s_s)
        t_s[...] = jnp.zeros_like(t_s)
        g_s[...] = jnp.zeros_like(g_s)

    m_old = m_s[...]
    mc = jnp.max(x, axis=0, keepdims=True)
    m_new = jnp.maximum(m_old, mc)
    corr = jnp.exp(m_old - m_new)
    s_new = s_s[...] * corr + jnp.sum(jnp.exp(x - m_new), axis=0, keepdims=True)
    m_s[...] = m_new
    s_s[...] = s_new
    t_s[...] += jnp.sum(x, axis=0, keepdims=True)

    rows = jax.lax.broadcasted_iota(jnp.int32, x.shape, 0) + i * ck
    hit = rows == lab
    g_new = g_s[...] + jnp.sum(jnp.where(hit, x, 0.0), axis=0, keepdims=True)
    g_s[...] = g_new

    @pl.when(i == nb - 1)
    def _fin():
        k = nb * ck
        lse = m_new + jnp.log(s_new)
        per = lse - (1.0 - _EPS) * g_new - (_EPS / k) * t_s[...]
        mask = (lab != _IGNORE).astype(jnp.float32)
        num_ref[...] = jnp.sum(per * mask).reshape(1, 1)
        den_ref[...] = jnp.sum(mask).reshape(1, 1)


@functools.partial(jax.jit, static_argnames=("ck",))
def _ce_loss(preds, labels, ck=4000):
    r, k = preds.shape
    pt = preds.T
    nb = k // ck
    lab2 = labels.astype(jnp.int32).reshape(1, r)
    num, den = pl.pallas_call(
        functools.partial(_tct_body, ck=ck, nb=nb),
        grid=(nb,),
        in_specs=[
            pl.BlockSpec((ck, r), lambda i: (i, 0)),
            pl.BlockSpec((1, r), lambda i: (0, 0)),
        ],
        out_specs=[
            pl.BlockSpec((1, 1), lambda i: (0, 0)),
            pl.BlockSpec((1, 1), lambda i: (0, 0)),
        ],
        out_shape=[
            jax.ShapeDtypeStruct((1, 1), jnp.float32),
            jax.ShapeDtypeStruct((1, 1), jnp.float32),
        ],
        scratch_shapes=[
            pltpu.VMEM((1, r), jnp.float32),
            pltpu.VMEM((1, r), jnp.float32),
            pltpu.VMEM((1, r), jnp.float32),
            pltpu.VMEM((1, r), jnp.float32),
        ],
    )(pt, lab2)
    return num[0, 0] / den[0, 0]


def kernel(preds, labels):
    return _ce_loss(preds, labels)
